# Initial kernel scaffold; baseline (speedup 1.0000x reference)
import jax
import jax.numpy as jnp
from jax.experimental import pallas as pl


def kernel(x, edge_index, y, sv_tri_a, sv_local_a, sv_tri_b, sv_local_b, W_enc1, b_enc1, W_enc2, b_enc2, codebook, Wd1, bd1, Wd2, bd2, Wd3, bd3):
    raise NotImplementedError("write your pallas kernel here")


# speedup vs baseline: 5.9195x; 5.9195x over previous
import functools

import jax
import jax.numpy as jnp
from jax import lax
from jax.experimental import pallas as pl
from jax.experimental.pallas import tpu as pltpu
from jax.experimental.pallas import tpu_sc as plsc

N = 10000
E = 320000
P = 30000
D_IN = 9
D_LAT = 128
K_CODE = 512
D_HID = 256
COMMIT = 1.0
DIV_W = 0.1
CONS_W = 1.0

NC = 2
NS = 16
NW = NC * NS
L = 16

N_PAD = 10240
ROWS_PER_TILE = N_PAD // NS
CH = 128
E_PER_W = 10112
N_CHUNKS = E_PER_W // CH
E_PAD = NW * E_PER_W
P_PAD = 30720
P_PER_W = P_PAD // NW
BLK = 1024
GRID = N_PAD // BLK


def _seg_sum_body(ndim, x_hbm, src_hbm, dst_hbm, zeros_hbm, out_hbm,
                  idx_s, idx_d, rows, acc, sem):
    c = lax.axis_index("c")
    s = lax.axis_index("s")
    w = c * NS + s
    r0 = s * ROWS_PER_TILE
    pltpu.sync_copy(zeros_hbm.at[pl.ds(r0, ROWS_PER_TILE)],
                    acc.at[pl.ds(r0, ROWS_PER_TILE)])
    plsc.subcore_barrier()

    def step(i, carry):
        base = w * E_PER_W + i * CH
        pltpu.sync_copy(src_hbm.at[pl.ds(base, CH)], idx_s)
        pltpu.sync_copy(dst_hbm.at[pl.ds(base, CH)], idx_d)
        pltpu.async_copy(x_hbm.at[idx_s], rows, sem).wait()
        pltpu.sync_copy(rows, acc.at[idx_d], add=True)
        return carry

    lax.fori_loop(0, N_CHUNKS, step, 0)
    plsc.subcore_barrier()
    pltpu.sync_copy(acc.at[pl.ds(r0, ROWS_PER_TILE)],
                    out_hbm.at[c, pl.ds(r0, ROWS_PER_TILE)])


def _make_seg_sum(ndim):
    mesh = plsc.VectorSubcoreMesh(core_axis_name="c", subcore_axis_name="s",
                                  num_cores=NC, num_subcores=NS)
    return pl.kernel(
        functools.partial(_seg_sum_body, ndim),
        out_type=jax.ShapeDtypeStruct((NC, N_PAD, ndim), jnp.float32),
        mesh=mesh,
        scratch_types=[
            pltpu.VMEM((CH,), jnp.int32),
            pltpu.VMEM((CH,), jnp.int32),
            pltpu.VMEM((CH, ndim), jnp.float32),
            pltpu.VMEM_SHARED((N_PAD, ndim), jnp.float32),
            pltpu.SemaphoreType.DMA,
        ],
        compiler_params=pltpu.CompilerParams(use_tc_tiling_on_sc=False),
    )


def _cons_body(rec_hbm, ta_hbm, la_hbm, tb_hbm, lb_hbm, out_hbm,
               rec_v, ta_v, la_v, tb_v, lb_v, acc_v):
    c = lax.axis_index("c")
    s = lax.axis_index("s")
    w = c * NS + s
    base = w * P_PER_W
    pltpu.sync_copy(rec_hbm, rec_v)
    pltpu.sync_copy(ta_hbm.at[pl.ds(base, P_PER_W)], ta_v)
    pltpu.sync_copy(la_hbm.at[pl.ds(base, P_PER_W)], la_v)
    pltpu.sync_copy(tb_hbm.at[pl.ds(base, P_PER_W)], tb_v)
    pltpu.sync_copy(lb_hbm.at[pl.ds(base, P_PER_W)], lb_v)

    def step(t, acc):
        ta = ta_v[pl.ds(t * L, L)]
        la = la_v[pl.ds(t * L, L)]
        tb = tb_v[pl.ds(t * L, L)]
        lb = lb_v[pl.ds(t * L, L)]
        fa = ta * 9 + la * 3
        fb = tb * 9 + lb * 3
        for j in range(3):
            av = plsc.load_gather(rec_v, [fa + j])
            bv = plsc.load_gather(rec_v, [fb + j])
            d = av - bv
            acc = acc + d * d
        return acc

    acc = lax.fori_loop(0, P_PER_W // L, step, jnp.zeros((L,), jnp.float32))
    acc_v[...] = acc
    pltpu.sync_copy(acc_v, out_hbm.at[w])


def _make_cons_kernel():
    return pl.kernel(
        _cons_body,
        out_type=jax.ShapeDtypeStruct((NW, L), jnp.float32),
        mesh=plsc.VectorSubcoreMesh(core_axis_name="c", subcore_axis_name="s",
                                    num_cores=NC, num_subcores=NS),
        scratch_types=[
            pltpu.VMEM((N_PAD * 9,), jnp.float32),
            pltpu.VMEM((P_PER_W,), jnp.int32),
            pltpu.VMEM((P_PER_W,), jnp.int32),
            pltpu.VMEM((P_PER_W,), jnp.int32),
            pltpu.VMEM((P_PER_W,), jnp.int32),
            pltpu.VMEM((L,), jnp.float32),
        ],
        compiler_params=pltpu.CompilerParams(use_tc_tiling_on_sc=False,
                                             needs_layout_passes=False),
    )


def _enc1_body(xa_ref, m1_ref, w1_ref, b1_ref, h_ref):
    m1 = m1_ref[0] + m1_ref[1]
    deg = jnp.maximum(m1[:, 9:10], 1.0)
    a = xa_ref[...] + m1 / deg
    h = jnp.maximum(jnp.dot(a, w1_ref[...], preferred_element_type=jnp.float32)
                    + b1_ref[...], 0.0)
    row = pl.program_id(0) * BLK + lax.broadcasted_iota(jnp.int32, (BLK, 1), 0)
    h_ref[...] = jnp.where(row < N, h, 0.0)


def _main_body(h_ref, m1_ref, m2_ref, y_ref, w2_ref, b2_ref, cbt_ref, c2_ref,
               cb_ref, wd1_ref, bd1_ref, wd2_ref, bd2_ref, wd3_ref, bd3_ref,
               rec_ref, idx_ref, cnt_ref, misc_ref):
    m1 = m1_ref[0] + m1_ref[1]
    deg = jnp.maximum(m1[:, 9:10], 1.0)
    m2 = m2_ref[0] + m2_ref[1]
    h = h_ref[...]
    ze = jnp.dot(h + m2 / deg, w2_ref[...],
                 preferred_element_type=jnp.float32) + b2_ref[...]
    z2 = jnp.sum(ze * ze, axis=1, keepdims=True)
    d = z2 - 2.0 * jnp.dot(ze, cbt_ref[...],
                           preferred_element_type=jnp.float32) + c2_ref[...]
    dmin = jnp.min(d, axis=1, keepdims=True)
    iot = lax.broadcasted_iota(jnp.int32, (BLK, K_CODE), 1)
    idx = jnp.min(jnp.where(d == dmin, iot, K_CODE), axis=1)
    onehot = (iot == idx[:, None]).astype(jnp.float32)
    zq = jnp.dot(onehot, cb_ref[...], preferred_element_type=jnp.float32)
    zq = ze + (zq - ze)
    h1 = jnp.maximum(jnp.dot(zq, wd1_ref[...],
                             preferred_element_type=jnp.float32) + bd1_ref[...], 0.0)
    h2 = jnp.maximum(jnp.dot(h1, wd2_ref[...],
                             preferred_element_type=jnp.float32) + bd2_ref[...], 0.0)
    rec = jnp.dot(h2, wd3_ref[...], preferred_element_type=jnp.float32) + bd3_ref[...]
    rec_ref[...] = rec
    idx_ref[...] = idx[None, None, :]
    row = pl.program_id(0) * BLK + lax.broadcasted_iota(jnp.int32, (BLK, 1), 0)
    real = (row < N).astype(jnp.float32)
    cnt_ref[...] = jnp.sum(onehot * real, axis=0)[None, None, :]
    abs_sum = jnp.sum(jnp.abs(rec - y_ref[...]) * real)
    dzq = ze - zq
    vq_sum = jnp.sum(jnp.sum(dzq * dzq, axis=1, keepdims=True) * real)
    lane = lax.broadcasted_iota(jnp.int32, (1, 128), 1)
    misc_ref[...] = (jnp.where(lane == 0, abs_sum, 0.0)
                     + jnp.where(lane == 1, vq_sum, 0.0))[None]


def _final_body(misc_ref, cnt_ref, cons_ref, out_ref):
    misc = jnp.sum(misc_ref[...], axis=(0, 1))
    abs_sum = misc[0]
    vq_sum = misc[1]
    counts = jnp.sum(cnt_ref[...], axis=(0, 1))
    avg_p = counts * (1.0 / N)
    div = jnp.sum(avg_p * jnp.log(avg_p + 1e-10)) + jnp.log(jnp.float32(K_CODE))
    cons = jnp.sum(cons_ref[...]) / (P * 3.0)
    recon_loss = abs_sum / (N * D_IN)
    cl = vq_sum / (N * D_LAT)
    vq_loss = cl + COMMIT * cl
    total = recon_loss + (vq_loss + DIV_W * div) + CONS_W * cons
    lane = lax.broadcasted_iota(jnp.int32, (1, 128), 1)
    out = jnp.where(lane == 0, total, 0.0)
    out = out + jnp.where(lane == 1, recon_loss, 0.0)
    out = out + jnp.where(lane == 2, vq_loss, 0.0)
    out = out + jnp.where(lane == 3, div, 0.0)
    out = out + jnp.where(lane == 4, cons, 0.0)
    out_ref[...] = out


def kernel(x, edge_index, y, sv_tri_a, sv_local_a, sv_tri_b, sv_local_b,
           W_enc1, b_enc1, W_enc2, b_enc2, codebook, Wd1, bd1, Wd2, bd2, Wd3, bd3):
    f32 = jnp.float32
    x_aug = jnp.concatenate(
        [x, jnp.ones((N, 1), f32), jnp.zeros((N, 6), f32)], axis=1)
    x_aug = jnp.pad(x_aug, ((0, N_PAD - N), (0, 0)))
    src_pad = jnp.concatenate(
        [edge_index[0], jnp.full((E_PAD - E,), N, jnp.int32)])
    dst_pad = jnp.concatenate(
        [edge_index[1], jnp.full((E_PAD - E,), N, jnp.int32)])
    zeros16 = jnp.zeros((N_PAD, 16), f32)
    zerosD = jnp.zeros((N_PAD, D_LAT), f32)
    y_pad = jnp.pad(y, ((0, N_PAD - N), (0, 0)))
    W1p = jnp.pad(W_enc1, ((0, 16 - D_IN), (0, 0)))
    cbT = codebook.T
    c2 = jnp.sum(codebook ** 2, axis=1)[None, :]
    ta_pad = jnp.pad(sv_tri_a, (0, P_PAD - P))
    la_pad = jnp.pad(sv_local_a, (0, P_PAD - P))
    tb_pad = jnp.pad(sv_tri_b, (0, P_PAD - P))
    lb_pad = jnp.pad(sv_local_b, (0, P_PAD - P))

    m1_parts = _make_seg_sum(16)(x_aug, src_pad, dst_pad, zeros16)

    h_pad = pl.pallas_call(
        _enc1_body,
        grid=(GRID,),
        in_specs=[
            pl.BlockSpec((BLK, 16), lambda i: (i, 0)),
            pl.BlockSpec((NC, BLK, 16), lambda i: (0, i, 0)),
            pl.BlockSpec((16, D_LAT), lambda i: (0, 0)),
            pl.BlockSpec((1, D_LAT), lambda i: (0, 0)),
        ],
        out_specs=pl.BlockSpec((BLK, D_LAT), lambda i: (i, 0)),
        out_shape=jax.ShapeDtypeStruct((N_PAD, D_LAT), f32),
    )(x_aug, m1_parts, W1p, b_enc1[None, :])

    m2_parts = _make_seg_sum(D_LAT)(h_pad, src_pad, dst_pad, zerosD)

    recon_pad, idx2d, cnts, misc = pl.pallas_call(
        _main_body,
        grid=(GRID,),
        in_specs=[
            pl.BlockSpec((BLK, D_LAT), lambda i: (i, 0)),
            pl.BlockSpec((NC, BLK, 16), lambda i: (0, i, 0)),
            pl.BlockSpec((NC, BLK, D_LAT), lambda i: (0, i, 0)),
            pl.BlockSpec((BLK, D_IN), lambda i: (i, 0)),
            pl.BlockSpec((D_LAT, D_LAT), lambda i: (0, 0)),
            pl.BlockSpec((1, D_LAT), lambda i: (0, 0)),
            pl.BlockSpec((D_LAT, K_CODE), lambda i: (0, 0)),
            pl.BlockSpec((1, K_CODE), lambda i: (0, 0)),
            pl.BlockSpec((K_CODE, D_LAT), lambda i: (0, 0)),
            pl.BlockSpec((D_LAT, D_HID), lambda i: (0, 0)),
            pl.BlockSpec((1, D_HID), lambda i: (0, 0)),
            pl.BlockSpec((D_HID, D_HID), lambda i: (0, 0)),
            pl.BlockSpec((1, D_HID), lambda i: (0, 0)),
            pl.BlockSpec((D_HID, D_IN), lambda i: (0, 0)),
            pl.BlockSpec((1, D_IN), lambda i: (0, 0)),
        ],
        out_specs=[
            pl.BlockSpec((BLK, D_IN), lambda i: (i, 0)),
            pl.BlockSpec((1, 1, BLK), lambda i: (i, 0, 0)),
            pl.BlockSpec((1, 1, K_CODE), lambda i: (i, 0, 0)),
            pl.BlockSpec((1, 1, 128), lambda i: (i, 0, 0)),
        ],
        out_shape=[
            jax.ShapeDtypeStruct((N_PAD, D_IN), f32),
            jax.ShapeDtypeStruct((GRID, 1, BLK), jnp.int32),
            jax.ShapeDtypeStruct((GRID, 1, K_CODE), f32),
            jax.ShapeDtypeStruct((GRID, 1, 128), f32),
        ],
    )(h_pad, m1_parts, m2_parts, y_pad, W_enc2, b_enc2[None, :], cbT, c2,
      codebook, Wd1, bd1[None, :], Wd2, bd2[None, :], Wd3, bd3[None, :])

    cons_parts = _make_cons_kernel()(recon_pad.reshape(N_PAD * 9), ta_pad,
                                     la_pad, tb_pad, lb_pad)

    scal = pl.pallas_call(
        _final_body,
        in_specs=[
            pl.BlockSpec((GRID, 1, 128), lambda: (0, 0, 0)),
            pl.BlockSpec((GRID, 1, K_CODE), lambda: (0, 0, 0)),
            pl.BlockSpec((NW, L), lambda: (0, 0)),
        ],
        out_specs=pl.BlockSpec((1, 128), lambda: (0, 0)),
        out_shape=jax.ShapeDtypeStruct((1, 128), f32),
    )(misc, cnts, cons_parts)

    recon = recon_pad[:N]
    indices = idx2d.reshape(N_PAD)[:N]
    total_loss = scal[0, 0]
    recon_loss = scal[0, 1]
    vq_loss = scal[0, 2]
    diversity_loss = scal[0, 3]
    consistency_loss = scal[0, 4]
    return (recon, total_loss, recon_loss, vq_loss, diversity_loss,
            consistency_loss, indices)
